# baseline (speedup 1.0000x reference)
import jax
import jax.numpy as jnp
from jax.experimental import pallas as pl
from jax.experimental.pallas import tpu as pltpu


def _gcn_block_kernel(x_ref, w_ref, b_ref, adj_ref, out_ref, supp_ref):
    @pl.when(pl.program_id(0) == 0)
    def _():
        supp_ref[...] = jax.nn.relu(
            jnp.dot(x_ref[...], w_ref[...], preferred_element_type=jnp.float32)
        )

    out_ref[...] = (
        jnp.dot(adj_ref[...], supp_ref[...], preferred_element_type=jnp.float32)
        + b_ref[...]
    )


def kernel(input, adj, gn_func, nn_func, weight, bias):
    n, d_in = input.shape
    d_out = weight.shape[1]
    bm = 80

    out = pl.pallas_call(
        _gcn_block_kernel,
        grid=(pl.cdiv(n, bm),),
        in_specs=[
            pl.BlockSpec((n, d_in), lambda i: (0, 0)),
            pl.BlockSpec((d_in, d_out), lambda i: (0, 0)),
            pl.BlockSpec((1, d_out), lambda i: (0, 0)),
            pl.BlockSpec((bm, n), lambda i: (i, 0)),
        ],
        out_specs=pl.BlockSpec((bm, d_out), lambda i: (i, 0)),
        out_shape=jax.ShapeDtypeStruct((n, d_out), jnp.float32),
        scratch_shapes=[pltpu.VMEM((n, d_out), jnp.float32)],
    )(input, weight, bias.reshape(1, d_out), adj)
    return out


# speedup vs baseline: 1.3657x; 1.3657x over previous
import jax
import jax.numpy as jnp
from jax.experimental import pallas as pl
from jax.experimental.pallas import tpu as pltpu


def _gcn_block_kernel(x_ref, w_ref, b_ref, adj_ref, out_ref, supp_ref):
    @pl.when(pl.program_id(0) == 0)
    def _():
        supp_ref[...] = jax.nn.relu(
            jnp.dot(x_ref[...], w_ref[...], preferred_element_type=jnp.float32)
        )

    out_ref[...] = (
        jnp.dot(adj_ref[...], supp_ref[...], preferred_element_type=jnp.float32)
        + b_ref[...]
    )


def kernel(input, adj, gn_func, nn_func, weight, bias):
    n, d_in = input.shape
    d_out = weight.shape[1]
    bm = 200

    out = pl.pallas_call(
        _gcn_block_kernel,
        grid=(pl.cdiv(n, bm),),
        in_specs=[
            pl.BlockSpec((n, d_in), lambda i: (0, 0)),
            pl.BlockSpec((d_in, d_out), lambda i: (0, 0)),
            pl.BlockSpec((1, d_out), lambda i: (0, 0)),
            pl.BlockSpec((bm, n), lambda i: (i, 0)),
        ],
        out_specs=pl.BlockSpec((bm, d_out), lambda i: (i, 0)),
        out_shape=jax.ShapeDtypeStruct((n, d_out), jnp.float32),
        scratch_shapes=[pltpu.VMEM((n, d_out), jnp.float32)],
    )(input, weight, bias.reshape(1, d_out), adj)
    return out
